# baseline (speedup 1.0000x reference)
import functools

import jax
import jax.numpy as jnp
from jax import lax
from jax.experimental import pallas as pl
from jax.experimental.pallas import tpu as pltpu
from jax.experimental.pallas import tpu_sc as plsc

_B = 16384
_F = 26
_V = 1000000
_D = 13

_NC = 2
_NS = 16
_NW = _NC * _NS
_BPW = _B // _NW
_CH = 128
_CPF = _BPW // _CH
_EPW = _F * _BPW
_GPW = _BPW // 16


def _sc_embed_linear(tables, idx_flat, dense_flat, wb):
    mesh = plsc.VectorSubcoreMesh(core_axis_name="c", subcore_axis_name="s")

    @functools.partial(
        pl.kernel,
        mesh=mesh,
        compiler_params=pltpu.CompilerParams(needs_layout_passes=False),
        out_type=jax.ShapeDtypeStruct((_B,), jnp.float32),
        scratch_types=[
            pltpu.VMEM((_EPW,), jnp.int32),
            pltpu.VMEM((_EPW,), jnp.int32),
            pltpu.VMEM((_EPW,), jnp.int32),
            pltpu.VMEM((_EPW,), jnp.int32),
            pltpu.VMEM((_D * _BPW,), jnp.float32),
            pltpu.VMEM((_F + _D + 1, 16), jnp.float32),
            pltpu.VMEM((_BPW,), jnp.float32),
            pltpu.SemaphoreType.DMA,
        ],
    )
    def k(*refs):
        tbl = refs[:_F]
        (idx_hbm, dense_hbm, wb_hbm, out_hbm,
         idx_v, idx_fm, widx_fm, g_v, dense_v, wb_v, out_v, sem) = refs[_F:]
        wid = lax.axis_index("s") * _NC + lax.axis_index("c")

        pltpu.sync_copy(idx_hbm.at[pl.ds(wid * _EPW, _EPW)], idx_v)

        iota = lax.iota(jnp.int32, 16)
        iota_f = iota * _F
        iota_d = iota * _D

        def xpose(s, carry):
            base = s * (16 * _F)
            for f in range(_F):
                v = plsc.load_gather(idx_v, [iota_f + (base + f)])
                idx_fm[pl.ds(f * _BPW + s * 16, 16)] = v
                widx_fm[pl.ds(f * _BPW + s * 16, 16)] = (
                    lax.shift_right_logical(v, 1))
            return carry
        lax.fori_loop(0, _GPW, xpose, 0)

        for f in range(_F):
            tref = tbl[f]

            def fire(c, carry, tref=tref, f=f):
                o = f * _BPW + c * _CH
                pltpu.make_async_copy(
                    tref.at[widx_fm.at[pl.ds(o, _CH)]],
                    g_v.at[pl.ds(o, _CH)], sem).start()
                return carry
            lax.fori_loop(0, _CPF, fire, 0)

        pltpu.sync_copy(dense_hbm.at[pl.ds(wid * _D * _BPW, _D * _BPW)],
                        dense_v)
        pltpu.sync_copy(wb_hbm, wb_v)

        def dense_part(s, carry):
            acc = wb_v[_F + _D]
            dbase = s * (16 * _D)
            for d in range(_D):
                v = plsc.load_gather(dense_v, [iota_d + (dbase + d)])
                acc = acc + v * wb_v[_F + d]
            out_v[pl.ds(s * 16, 16)] = acc
            return carry
        lax.fori_loop(0, _GPW, dense_part, 0)

        def drain(j, carry):
            pltpu.make_async_copy(
                tbl[0].at[widx_fm.at[pl.ds(0, _CH)]],
                g_v.at[pl.ds(0, _CH)], sem).wait()
            return carry
        lax.fori_loop(0, _F * _CPF, drain, 0)

        one16 = jnp.full((16,), 1, jnp.int32)
        sixteen16 = jnp.full((16,), 16, jnp.int32)
        himask = jnp.full((16,), -65536, jnp.int32)

        def emb_part(s, carry):
            acc = out_v[pl.ds(s * 16, 16)]
            for f in range(_F):
                w_u = g_v[pl.ds(f * _BPW + s * 16, 16)]
                iv = idx_fm[pl.ds(f * _BPW + s * 16, 16)]
                sh = (one16 - (iv & one16)) * sixteen16
                bits = lax.shift_left(w_u, sh) & himask
                acc = acc + plsc.bitcast(bits, jnp.float32) * wb_v[f]
            out_v[pl.ds(s * 16, 16)] = acc
            return carry
        lax.fori_loop(0, _GPW, emb_part, 0)

        pltpu.sync_copy(out_v, out_hbm.at[pl.ds(wid * _BPW, _BPW)])

    return k(*tables, idx_flat, dense_flat, wb)


def kernel(sparse_idx, dense, emb_tables, fc_w, fc_b):
    tables = [
        lax.bitcast_convert_type(
            lax.squeeze(lax.slice_in_dim(emb_tables, f, f + 1, axis=0),
                        (0, 2)).astype(jnp.bfloat16).reshape(_V // 2, 2),
            jnp.int32)
        for f in range(_F)]
    idx_flat = sparse_idx.reshape(_B * _F)
    dense_flat = dense.reshape(_B * _D)
    wb = jnp.broadcast_to(
        jnp.concatenate([fc_w.reshape(-1), fc_b]).reshape(_F + _D + 1, 1),
        (_F + _D + 1, 16))
    out = _sc_embed_linear(tables, idx_flat, dense_flat, wb)
    return out.reshape(_B, 1)


# speedup vs baseline: 19.7436x; 19.7436x over previous
import functools

import jax
import jax.numpy as jnp
from jax import lax
from jax.experimental import pallas as pl
from jax.experimental.pallas import tpu as pltpu
from jax.experimental.pallas import tpu_sc as plsc

_B = 16384
_F = 26
_V = 1000000
_D = 13

_NC = 2
_NS = 16
_NW = _NC * _NS
_BPW = _B // _NW
_CH = 128
_CPF = _BPW // _CH
_EPW = _F * _BPW
_GPW = _BPW // 16


def _sc_embed_linear(tables, idx_flat, dense_flat, wb):
    mesh = plsc.VectorSubcoreMesh(core_axis_name="c", subcore_axis_name="s")

    @functools.partial(
        pl.kernel,
        mesh=mesh,
        compiler_params=pltpu.CompilerParams(needs_layout_passes=False),
        out_type=jax.ShapeDtypeStruct((_B,), jnp.float32),
        scratch_types=[
            pltpu.VMEM((_EPW,), jnp.int32),
            pltpu.VMEM((_EPW,), jnp.int32),
            pltpu.VMEM((_EPW,), jnp.float32),
            pltpu.VMEM((_D * _BPW,), jnp.float32),
            pltpu.VMEM((_F + _D + 1, 16), jnp.float32),
            pltpu.VMEM((_BPW,), jnp.float32),
            pltpu.SemaphoreType.DMA,
        ],
    )
    def k(*refs):
        tbl = refs[:_F]
        (idx_hbm, dense_hbm, wb_hbm, out_hbm,
         idx_v, idx_fm, g_v, dense_v, wb_v, out_v, sem) = refs[_F:]
        wid = lax.axis_index("s") * _NC + lax.axis_index("c")

        pltpu.sync_copy(idx_hbm.at[pl.ds(wid * _EPW, _EPW)], idx_v)

        iota = lax.iota(jnp.int32, 16)
        iota_f = iota * _F
        iota_d = iota * _D

        def xpose(s, carry):
            base = s * (16 * _F)
            for f in range(_F):
                v = plsc.load_gather(idx_v, [iota_f + (base + f)])
                idx_fm[pl.ds(f * _BPW + s * 16, 16)] = v
            return carry
        lax.fori_loop(0, _GPW, xpose, 0)

        for f in range(_F):
            tref = tbl[f]

            def fire(c, carry, tref=tref, f=f):
                o = f * _BPW + c * _CH
                pltpu.make_async_copy(
                    tref.at[idx_fm.at[pl.ds(o, _CH)]],
                    g_v.at[pl.ds(o, _CH)], sem).start()
                return carry
            lax.fori_loop(0, _CPF, fire, 0)

        pltpu.sync_copy(dense_hbm.at[pl.ds(wid * _D * _BPW, _D * _BPW)],
                        dense_v)
        pltpu.sync_copy(wb_hbm, wb_v)

        def dense_part(s, carry):
            acc = wb_v[_F + _D]
            dbase = s * (16 * _D)
            for d in range(_D):
                v = plsc.load_gather(dense_v, [iota_d + (dbase + d)])
                acc = acc + v * wb_v[_F + d]
            out_v[pl.ds(s * 16, 16)] = acc
            return carry
        lax.fori_loop(0, _GPW, dense_part, 0)

        def drain(j, carry):
            pltpu.make_async_copy(
                tbl[0].at[idx_fm.at[pl.ds(0, _CH)]],
                g_v.at[pl.ds(0, _CH)], sem).wait()
            return carry
        lax.fori_loop(0, _F * _CPF, drain, 0)

        def emb_part(s, carry):
            acc = out_v[pl.ds(s * 16, 16)]
            for f in range(_F):
                v = g_v[pl.ds(f * _BPW + s * 16, 16)]
                acc = acc + v * wb_v[f]
            out_v[pl.ds(s * 16, 16)] = acc
            return carry
        lax.fori_loop(0, _GPW, emb_part, 0)

        pltpu.sync_copy(out_v, out_hbm.at[pl.ds(wid * _BPW, _BPW)])

    return k(*tables, idx_flat, dense_flat, wb)


def kernel(sparse_idx, dense, emb_tables, fc_w, fc_b):
    tables = [
        lax.squeeze(lax.slice_in_dim(emb_tables, f, f + 1, axis=0), (0, 2))
        for f in range(_F)]
    idx_flat = sparse_idx.reshape(_B * _F)
    dense_flat = dense.reshape(_B * _D)
    wb = jnp.broadcast_to(
        jnp.concatenate([fc_w.reshape(-1), fc_b]).reshape(_F + _D + 1, 1),
        (_F + _D + 1, 16))
    out = _sc_embed_linear(tables, idx_flat, dense_flat, wb)
    return out.reshape(_B, 1)
